# baseline (speedup 1.0000x reference)
import functools

import jax
import jax.numpy as jnp
from jax import lax
from jax.experimental import pallas as pl
from jax.experimental.pallas import tpu as pltpu, tpu_sc as plsc

_B = 16384
_IN = 512
_OUT = 128
_FAN_IN = 4
_NEW_IN = 5

_L = 16
_NC = 2
_NS = 16
_NW = _NC * _NS
_B_SC = 4096
_RPW = _B_SC // _NW
_R = 64
_NCHUNK = _RPW // _R
_UNROLL = 4

_B_TC = _B - _B_SC
_TB = 2048


def _splat_i32(v):
    return jnp.full((16,), v, dtype=jnp.int32)


def _sc_body(x_hbm, w_hbm, b_hbm, imask_hbm, out_hbm,
             xbuf0, xbuf1, obuf0, obuf1, wv, bv, imv, insems, outsems):
    xbufs = (xbuf0, xbuf1)
    obufs = (obuf0, obuf1)
    wid = lax.axis_index("s") * _NC + lax.axis_index("c")
    base = wid * _RPW

    pltpu.sync_copy(w_hbm, wv)
    pltpu.sync_copy(b_hbm, bv)
    pltpu.sync_copy(imask_hbm, imv)

    iota = lax.broadcasted_iota(jnp.int32, (16,), 0)

    consts = []
    for j in range(_OUT // _L):
        oidx = iota + _L * j
        c = plsc.load_gather(wv, [oidx, _splat_i32(0)]) + bv[pl.ds(_L * j, _L)]
        idx_k = [plsc.load_gather(imv, [oidx, _splat_i32(k)])
                 for k in range(_FAN_IN)]
        w_k = [plsc.load_gather(wv, [oidx, _splat_i32(k + 1)])
               for k in range(_FAN_IN)]
        consts.append((c, idx_k, w_k))

    def start_in(ci, b):
        pltpu.async_copy(
            x_hbm.at[pl.ds(base + ci * _R, _R)],
            xbufs[b], insems.at[b])

    def wait_in(b):
        pltpu.make_async_copy(
            x_hbm.at[pl.ds(base, _R)],
            xbufs[b], insems.at[b]).wait()

    def start_out(ci, b):
        pltpu.async_copy(
            obufs[b],
            out_hbm.at[pl.ds(base + ci * _R, _R)],
            outsems.at[b])

    def wait_out(b):
        pltpu.make_async_copy(
            obufs[b],
            out_hbm.at[pl.ds(base, _R)],
            outsems.at[b]).wait()

    def compute(b):
        xbuf = xbufs[b]
        obuf = obufs[b]
        for j in range(_OUT // _L):
            c, idx_k, w_k = consts[j]

            @plsc.parallel_loop(0, _R, step=1, unroll=_UNROLL)
            def row_body(r, c=c, idx_k=idx_k, w_k=w_k, j=j):
                rv = jnp.full((16,), r, dtype=jnp.int32)
                acc = c
                for k in range(_FAN_IN):
                    xv = plsc.load_gather(xbuf, [rv, idx_k[k]])
                    acc = acc + xv * w_k[k]
                obuf[r, pl.ds(_L * j, _L)] = acc

    start_in(0, 0)
    start_in(1, 1)

    def loop_body(g, _):
        for b in range(2):
            ci = g * 2 + b
            wait_in(b)

            @pl.when(g > 0)
            def _():
                wait_out(b)

            compute(b)
            start_out(ci, b)

            @pl.when(ci + 2 < _NCHUNK)
            def _():
                start_in(ci + 2, b)
        return 0

    lax.fori_loop(0, _NCHUNK // 2, loop_body, 0)
    wait_out(0)
    wait_out(1)


def _sc_part(x_sc, W, b, imask):
    mesh = plsc.VectorSubcoreMesh(core_axis_name="c", subcore_axis_name="s")
    run = pl.kernel(
        _sc_body,
        mesh=mesh,
        out_type=jax.ShapeDtypeStruct((_B_SC, _OUT), jnp.float32),
        scratch_types=[
            pltpu.VMEM((_R, _IN), jnp.float32),
            pltpu.VMEM((_R, _IN), jnp.float32),
            pltpu.VMEM((_R, _OUT), jnp.float32),
            pltpu.VMEM((_R, _OUT), jnp.float32),
            pltpu.VMEM((_OUT, _NEW_IN), jnp.float32),
            pltpu.VMEM((_OUT,), jnp.float32),
            pltpu.VMEM((_OUT, _FAN_IN), jnp.int32),
            pltpu.SemaphoreType.DMA((2,)),
            pltpu.SemaphoreType.DMA((2,)),
        ],
        compiler_params=pltpu.CompilerParams(needs_layout_passes=False),
    )
    return run(x_sc, W, b, imask)


def _tc_kernel(x_ref, w_ref, b_ref, imask_ref, o_ref):
    w = w_ref[...]
    imask = imask_ref[...]
    iota = jax.lax.broadcasted_iota(jnp.int32, (1, 1, _IN), 2)
    eq = (imask[:, :, None] == iota).astype(jnp.float32)
    mt = jnp.sum(eq * w[:, 1:, None], axis=1)
    x = x_ref[...]
    y = jax.lax.dot_general(
        x, mt, (((1,), (1,)), ((), ())), preferred_element_type=jnp.float32
    )
    o_ref[...] = y + (w[:, 0] + b_ref[...])[None, :]


def _tc_part(x_tc, W, b, imask):
    grid = (_B_TC // _TB,)
    return pl.pallas_call(
        _tc_kernel,
        grid=grid,
        in_specs=[
            pl.BlockSpec((_TB, _IN), lambda i: (i, 0)),
            pl.BlockSpec((_OUT, _NEW_IN), lambda i: (0, 0)),
            pl.BlockSpec((_OUT,), lambda i: (0,)),
            pl.BlockSpec((_OUT, _FAN_IN), lambda i: (0, 0)),
        ],
        out_specs=pl.BlockSpec((_TB, _OUT), lambda i: (i, 0)),
        out_shape=jax.ShapeDtypeStruct((_B_TC, _OUT), jnp.float32),
    )(x_tc, W, b, imask)


@jax.jit
def kernel(x, W, b, imask, mask):
    del mask
    if _B_SC == _B:
        return _sc_part(x, W, b, imask)
    y_sc = _sc_part(x[:_B_SC], W, b, imask)
    y_tc = _tc_part(x[_B_SC:], W, b, imask)
    return jnp.concatenate([y_sc, y_tc], axis=0)


# speedup vs baseline: 1.0043x; 1.0043x over previous
import functools

import jax
import jax.numpy as jnp
from jax import lax
from jax.experimental import pallas as pl
from jax.experimental.pallas import tpu as pltpu, tpu_sc as plsc

_B = 16384
_IN = 512
_OUT = 128
_FAN_IN = 4
_NEW_IN = 5

_L = 16
_NC = 2
_NS = 16
_NW = _NC * _NS
_B_SC = 4096
_RPW = _B_SC // _NW
_R = 64
_NCHUNK = _RPW // _R
_UNROLL = 4

_B_TC = _B - _B_SC
_TB = 2048


def _splat_i32(v):
    return jnp.full((16,), v, dtype=jnp.int32)


def _sc_body(x_hbm, w_hbm, b_hbm, imask_hbm, out_hbm,
             xbuf0, xbuf1, obuf0, obuf1, wv, bv, imv, insems, outsems):
    xbufs = (xbuf0, xbuf1)
    obufs = (obuf0, obuf1)
    wid = lax.axis_index("s") * _NC + lax.axis_index("c")
    base = wid * _RPW

    pltpu.sync_copy(w_hbm, wv)
    pltpu.sync_copy(b_hbm, bv)
    pltpu.sync_copy(imask_hbm, imv)

    iota = lax.broadcasted_iota(jnp.int32, (16,), 0)

    def start_in(ci, b):
        pltpu.async_copy(
            x_hbm.at[pl.ds(base + ci * _R, _R)],
            xbufs[b], insems.at[b])

    def wait_in(b):
        pltpu.make_async_copy(
            x_hbm.at[pl.ds(base, _R)],
            xbufs[b], insems.at[b]).wait()

    def start_out(ci, b):
        pltpu.async_copy(
            obufs[b],
            out_hbm.at[pl.ds(base + ci * _R, _R)],
            outsems.at[b])

    def wait_out(b):
        pltpu.make_async_copy(
            obufs[b],
            out_hbm.at[pl.ds(base, _R)],
            outsems.at[b]).wait()

    def compute(b):
        xbuf = xbufs[b]
        obuf = obufs[b]

        def j_body(j, _):
            oidx = iota + _L * j
            c = (plsc.load_gather(wv, [oidx, _splat_i32(0)])
                 + plsc.load_gather(bv, [oidx]))
            idx_k = [plsc.load_gather(imv, [oidx, _splat_i32(k)])
                     for k in range(_FAN_IN)]
            w_k = [plsc.load_gather(wv, [oidx, _splat_i32(k + 1)])
                   for k in range(_FAN_IN)]

            @plsc.parallel_loop(0, _R, step=1, unroll=_UNROLL)
            def row_body(r):
                rv = jnp.full((16,), r, dtype=jnp.int32)
                acc = c
                for k in range(_FAN_IN):
                    xv = plsc.load_gather(xbuf, [rv, idx_k[k]])
                    acc = acc + xv * w_k[k]
                obuf[r, pl.ds(j * _L, _L)] = acc

            return 0

        lax.fori_loop(0, _OUT // _L, j_body, 0)

    start_in(0, 0)
    start_in(1, 1)

    def loop_body(g, _):
        for b in range(2):
            ci = g * 2 + b
            wait_in(b)

            @pl.when(g > 0)
            def _():
                wait_out(b)

            compute(b)
            start_out(ci, b)

            @pl.when(ci + 2 < _NCHUNK)
            def _():
                start_in(ci + 2, b)
        return 0

    lax.fori_loop(0, _NCHUNK // 2, loop_body, 0)
    wait_out(0)
    wait_out(1)


def _sc_part(x_sc, W, b, imask):
    mesh = plsc.VectorSubcoreMesh(core_axis_name="c", subcore_axis_name="s")
    run = pl.kernel(
        _sc_body,
        mesh=mesh,
        out_type=jax.ShapeDtypeStruct((_B_SC, _OUT), jnp.float32),
        scratch_types=[
            pltpu.VMEM((_R, _IN), jnp.float32),
            pltpu.VMEM((_R, _IN), jnp.float32),
            pltpu.VMEM((_R, _OUT), jnp.float32),
            pltpu.VMEM((_R, _OUT), jnp.float32),
            pltpu.VMEM((_OUT, _NEW_IN), jnp.float32),
            pltpu.VMEM((_OUT,), jnp.float32),
            pltpu.VMEM((_OUT, _FAN_IN), jnp.int32),
            pltpu.SemaphoreType.DMA((2,)),
            pltpu.SemaphoreType.DMA((2,)),
        ],
        compiler_params=pltpu.CompilerParams(needs_layout_passes=False),
    )
    return run(x_sc, W, b, imask)


def _tc_kernel(x_ref, w_ref, b_ref, imask_ref, o_ref):
    w = w_ref[...]
    imask = imask_ref[...]
    iota = jax.lax.broadcasted_iota(jnp.int32, (1, 1, _IN), 2)
    eq = (imask[:, :, None] == iota).astype(jnp.float32)
    mt = jnp.sum(eq * w[:, 1:, None], axis=1)
    x = x_ref[...]
    y = jax.lax.dot_general(
        x, mt, (((1,), (1,)), ((), ())), preferred_element_type=jnp.float32
    )
    o_ref[...] = y + (w[:, 0] + b_ref[...])[None, :]


def _tc_part(x_tc, W, b, imask):
    grid = (_B_TC // _TB,)
    return pl.pallas_call(
        _tc_kernel,
        grid=grid,
        in_specs=[
            pl.BlockSpec((_TB, _IN), lambda i: (i, 0)),
            pl.BlockSpec((_OUT, _NEW_IN), lambda i: (0, 0)),
            pl.BlockSpec((_OUT,), lambda i: (0,)),
            pl.BlockSpec((_OUT, _FAN_IN), lambda i: (0, 0)),
        ],
        out_specs=pl.BlockSpec((_TB, _OUT), lambda i: (i, 0)),
        out_shape=jax.ShapeDtypeStruct((_B_TC, _OUT), jnp.float32),
    )(x_tc, W, b, imask)


@jax.jit
def kernel(x, W, b, imask, mask):
    del mask
    if _B_SC == _B:
        return _sc_part(x, W, b, imask)
    y_sc = _sc_part(x[:_B_SC], W, b, imask)
    y_tc = _tc_part(x[_B_SC:], W, b, imask)
    return jnp.concatenate([y_sc, y_tc], axis=0)


# speedup vs baseline: 1.4817x; 1.4752x over previous
import functools

import jax
import jax.numpy as jnp
from jax import lax
from jax.experimental import pallas as pl
from jax.experimental.pallas import tpu as pltpu, tpu_sc as plsc

_B = 16384
_IN = 512
_OUT = 128
_FAN_IN = 4
_NEW_IN = 5

_L = 16
_NC = 2
_NS = 16
_NW = _NC * _NS
_B_SC = 4096
_RPW = _B_SC // _NW
_R = 64
_NCHUNK = _RPW // _R
_UNROLL = 4

_B_TC = _B - _B_SC
_TB = 2048


def _splat_i32(v):
    return jnp.full((16,), v, dtype=jnp.int32)


def _sc_body(x_hbm, w_hbm, b_hbm, imask_hbm, out_hbm,
             xbuf0, xbuf1, obuf0, obuf1, wv, bv, imv, insems, outsems):
    xbufs = (xbuf0, xbuf1)
    obufs = (obuf0, obuf1)
    wid = lax.axis_index("s") * _NC + lax.axis_index("c")
    base = wid * _RPW

    pltpu.sync_copy(w_hbm, wv)
    pltpu.sync_copy(b_hbm, bv)
    pltpu.sync_copy(imask_hbm, imv)

    iota = lax.broadcasted_iota(jnp.int32, (16,), 0)

    def start_in(ci, b):
        pltpu.async_copy(
            x_hbm.at[pl.ds(base + ci * _R, _R)],
            xbufs[b], insems.at[b])

    def wait_in(b):
        pltpu.make_async_copy(
            x_hbm.at[pl.ds(base, _R)],
            xbufs[b], insems.at[b]).wait()

    def start_out(ci, b):
        pltpu.async_copy(
            obufs[b],
            out_hbm.at[pl.ds(base + ci * _R, _R)],
            outsems.at[b])

    def wait_out(b):
        pltpu.make_async_copy(
            obufs[b],
            out_hbm.at[pl.ds(base, _R)],
            outsems.at[b]).wait()

    def compute(b):
        xbuf = xbufs[b]
        obuf = obufs[b]

        def j_body(j, _):
            oidx = iota + _L * j
            c = (plsc.load_gather(wv, [oidx, _splat_i32(0)])
                 + plsc.load_gather(bv, [oidx]))
            idx_k = [plsc.load_gather(imv, [oidx, _splat_i32(k)])
                     for k in range(_FAN_IN)]
            w_k = [plsc.load_gather(wv, [oidx, _splat_i32(k + 1)])
                   for k in range(_FAN_IN)]

            @plsc.parallel_loop(0, _R, step=1, unroll=_UNROLL)
            def row_body(r):
                rv = jnp.full((16,), r, dtype=jnp.int32)
                acc = c
                for k in range(_FAN_IN):
                    xv = plsc.load_gather(xbuf, [rv, idx_k[k]])
                    acc = acc + xv * w_k[k]
                obuf[r, pl.ds(j * _L, _L)] = acc

            return 0

        lax.fori_loop(0, _OUT // _L, j_body, 0)

    start_in(0, 0)
    start_in(1, 1)

    def loop_body(g, _):
        for b in range(2):
            ci = g * 2 + b
            wait_in(b)

            @pl.when(g > 0)
            def _():
                wait_out(b)

            compute(b)
            start_out(ci, b)

            @pl.when(ci + 2 < _NCHUNK)
            def _():
                start_in(ci + 2, b)
        return 0

    lax.fori_loop(0, _NCHUNK // 2, loop_body, 0)
    wait_out(0)
    wait_out(1)


def _sc_part(x_sc, W, b, imask):
    mesh = plsc.VectorSubcoreMesh(core_axis_name="c", subcore_axis_name="s")
    run = pl.kernel(
        _sc_body,
        mesh=mesh,
        out_type=jax.ShapeDtypeStruct((_B_SC, _OUT), jnp.float32),
        scratch_types=[
            pltpu.VMEM((_R, _IN), jnp.float32),
            pltpu.VMEM((_R, _IN), jnp.float32),
            pltpu.VMEM((_R, _OUT), jnp.float32),
            pltpu.VMEM((_R, _OUT), jnp.float32),
            pltpu.VMEM((_OUT, _NEW_IN), jnp.float32),
            pltpu.VMEM((_OUT,), jnp.float32),
            pltpu.VMEM((_OUT, _FAN_IN), jnp.int32),
            pltpu.SemaphoreType.DMA((2,)),
            pltpu.SemaphoreType.DMA((2,)),
        ],
        compiler_params=pltpu.CompilerParams(needs_layout_passes=False),
    )
    return run(x_sc, W, b, imask)


def _tc_kernel(x_ref, w_ref, b_ref, imask_ref, o_ref):
    w = w_ref[...]
    imask = imask_ref[...]
    iota = jax.lax.broadcasted_iota(jnp.int32, (1, 1, _IN), 2)
    eq = (imask[:, :, None] == iota).astype(jnp.float32)
    mt = jnp.sum(eq * w[:, 1:, None], axis=1)
    x = x_ref[...]
    y = jax.lax.dot_general(
        x, mt, (((1,), (1,)), ((), ())), preferred_element_type=jnp.float32
    )
    o_ref[...] = y + (w[:, 0] + b_ref[...])[None, :]


def _tc_part(x, W, b, imask):
    grid = (_B_TC // _TB,)
    off = _B_SC // _TB
    return pl.pallas_call(
        _tc_kernel,
        grid=grid,
        in_specs=[
            pl.BlockSpec((_TB, _IN), lambda i: (i + off, 0)),
            pl.BlockSpec((_OUT, _NEW_IN), lambda i: (0, 0)),
            pl.BlockSpec((_OUT,), lambda i: (0,)),
            pl.BlockSpec((_OUT, _FAN_IN), lambda i: (0, 0)),
        ],
        out_specs=pl.BlockSpec((_TB, _OUT), lambda i: (i, 0)),
        out_shape=jax.ShapeDtypeStruct((_B_TC, _OUT), jnp.float32),
    )(x, W, b, imask)


@jax.jit
def kernel(x, W, b, imask, mask):
    del mask
    if _B_SC == _B:
        return _sc_part(x, W, b, imask)
    y_sc = _sc_part(x, W, b, imask)
    y_tc = _tc_part(x, W, b, imask)
    return jnp.concatenate([y_sc, y_tc], axis=0)


# speedup vs baseline: 1.5931x; 1.0752x over previous
import functools

import jax
import jax.numpy as jnp
from jax import lax
from jax.experimental import pallas as pl
from jax.experimental.pallas import tpu as pltpu, tpu_sc as plsc

_B = 16384
_IN = 512
_OUT = 128
_FAN_IN = 4
_NEW_IN = 5

_L = 16
_NC = 2
_NS = 16
_NW = _NC * _NS
_B_SC = 2048
_RPW = _B_SC // _NW
_R = 32
_NCHUNK = _RPW // _R
_UNROLL = 4

_B_TC = _B - _B_SC
_TB = 2048


def _splat_i32(v):
    return jnp.full((16,), v, dtype=jnp.int32)


def _sc_body(x_hbm, w_hbm, b_hbm, imask_hbm, out_hbm,
             xbuf0, xbuf1, obuf0, obuf1, wv, bv, imv, insems, outsems):
    xbufs = (xbuf0, xbuf1)
    obufs = (obuf0, obuf1)
    wid = lax.axis_index("s") * _NC + lax.axis_index("c")
    base = wid * _RPW

    pltpu.sync_copy(w_hbm, wv)
    pltpu.sync_copy(b_hbm, bv)
    pltpu.sync_copy(imask_hbm, imv)

    iota = lax.broadcasted_iota(jnp.int32, (16,), 0)

    def start_in(ci, b):
        pltpu.async_copy(
            x_hbm.at[pl.ds(base + ci * _R, _R)],
            xbufs[b], insems.at[b])

    def wait_in(b):
        pltpu.make_async_copy(
            x_hbm.at[pl.ds(base, _R)],
            xbufs[b], insems.at[b]).wait()

    def start_out(ci, b):
        pltpu.async_copy(
            obufs[b],
            out_hbm.at[pl.ds(base + ci * _R, _R)],
            outsems.at[b])

    def wait_out(b):
        pltpu.make_async_copy(
            obufs[b],
            out_hbm.at[pl.ds(base, _R)],
            outsems.at[b]).wait()

    def compute(b):
        xbuf = xbufs[b]
        obuf = obufs[b]

        def j_body(j, _):
            oidx = iota + _L * j
            c = (plsc.load_gather(wv, [oidx, _splat_i32(0)])
                 + plsc.load_gather(bv, [oidx]))
            idx_k = [plsc.load_gather(imv, [oidx, _splat_i32(k)])
                     for k in range(_FAN_IN)]
            w_k = [plsc.load_gather(wv, [oidx, _splat_i32(k + 1)])
                   for k in range(_FAN_IN)]

            @plsc.parallel_loop(0, _R, step=1, unroll=_UNROLL)
            def row_body(r):
                rv = jnp.full((16,), r, dtype=jnp.int32)
                acc = c
                for k in range(_FAN_IN):
                    xv = plsc.load_gather(xbuf, [rv, idx_k[k]])
                    acc = acc + xv * w_k[k]
                obuf[r, pl.ds(j * _L, _L)] = acc

            return 0

        lax.fori_loop(0, _OUT // _L, j_body, 0)

    start_in(0, 0)
    start_in(1, 1)

    def loop_body(g, _):
        for b in range(2):
            ci = g * 2 + b
            wait_in(b)

            @pl.when(g > 0)
            def _():
                wait_out(b)

            compute(b)
            start_out(ci, b)

            @pl.when(ci + 2 < _NCHUNK)
            def _():
                start_in(ci + 2, b)
        return 0

    lax.fori_loop(0, _NCHUNK // 2, loop_body, 0)
    wait_out(0)
    wait_out(1)


def _sc_part(x_sc, W, b, imask):
    mesh = plsc.VectorSubcoreMesh(core_axis_name="c", subcore_axis_name="s")
    run = pl.kernel(
        _sc_body,
        mesh=mesh,
        out_type=jax.ShapeDtypeStruct((_B_SC, _OUT), jnp.float32),
        scratch_types=[
            pltpu.VMEM((_R, _IN), jnp.float32),
            pltpu.VMEM((_R, _IN), jnp.float32),
            pltpu.VMEM((_R, _OUT), jnp.float32),
            pltpu.VMEM((_R, _OUT), jnp.float32),
            pltpu.VMEM((_OUT, _NEW_IN), jnp.float32),
            pltpu.VMEM((_OUT,), jnp.float32),
            pltpu.VMEM((_OUT, _FAN_IN), jnp.int32),
            pltpu.SemaphoreType.DMA((2,)),
            pltpu.SemaphoreType.DMA((2,)),
        ],
        compiler_params=pltpu.CompilerParams(needs_layout_passes=False),
    )
    return run(x_sc, W, b, imask)


def _tc_kernel(x_ref, w_ref, b_ref, imask_ref, o_ref):
    w = w_ref[...]
    imask = imask_ref[...]
    iota = jax.lax.broadcasted_iota(jnp.int32, (1, 1, _IN), 2)
    eq = (imask[:, :, None] == iota).astype(jnp.float32)
    mt = jnp.sum(eq * w[:, 1:, None], axis=1)
    x = x_ref[...]
    y = jax.lax.dot_general(
        x, mt, (((1,), (1,)), ((), ())), preferred_element_type=jnp.float32
    )
    o_ref[...] = y + (w[:, 0] + b_ref[...])[None, :]


def _tc_part(x, W, b, imask):
    grid = (_B_TC // _TB,)
    off = _B_SC // _TB
    return pl.pallas_call(
        _tc_kernel,
        grid=grid,
        in_specs=[
            pl.BlockSpec((_TB, _IN), lambda i: (i + off, 0)),
            pl.BlockSpec((_OUT, _NEW_IN), lambda i: (0, 0)),
            pl.BlockSpec((_OUT,), lambda i: (0,)),
            pl.BlockSpec((_OUT, _FAN_IN), lambda i: (0, 0)),
        ],
        out_specs=pl.BlockSpec((_TB, _OUT), lambda i: (i, 0)),
        out_shape=jax.ShapeDtypeStruct((_B_TC, _OUT), jnp.float32),
    )(x, W, b, imask)


@jax.jit
def kernel(x, W, b, imask, mask):
    del mask
    if _B_SC == _B:
        return _sc_part(x, W, b, imask)
    y_sc = _sc_part(x, W, b, imask)
    y_tc = _tc_part(x, W, b, imask)
    return jnp.concatenate([y_sc, y_tc], axis=0)
